# baseline (speedup 1.0000x reference)
import functools

import jax
import jax.numpy as jnp
from jax.experimental import pallas as pl

_BLOCK = 2048


def _nearest_body(x_ref, c_ref, idx_ref, dist_ref):
    x = x_ref[...]
    c = c_ref[...]
    x2 = jnp.sum(x * x, axis=1, keepdims=True)
    c2 = jnp.sum(c * c, axis=1)[None, :]
    dots = jax.lax.dot_general(
        x, c, (((1,), (1,)), ((), ())),
        preferred_element_type=jnp.float32)
    d2 = jnp.maximum(x2 + c2 - 2.0 * dots, 0.0)
    idx_ref[...] = jnp.argmin(d2, axis=1).astype(jnp.int32)
    dist_ref[...] = jnp.sqrt(jnp.min(d2, axis=1))


@functools.partial(jax.jit, static_argnames=())
def kernel(X, codebook):
    n, d = X.shape
    k = codebook.shape[0]
    grid = n // _BLOCK
    idx, dist = pl.pallas_call(
        _nearest_body,
        grid=(grid,),
        in_specs=[
            pl.BlockSpec((_BLOCK, d), lambda i: (i, 0)),
            pl.BlockSpec((k, d), lambda i: (0, 0)),
        ],
        out_specs=[
            pl.BlockSpec((_BLOCK,), lambda i: (i,)),
            pl.BlockSpec((_BLOCK,), lambda i: (i,)),
        ],
        out_shape=[
            jax.ShapeDtypeStruct((n,), jnp.int32),
            jax.ShapeDtypeStruct((n,), jnp.float32),
        ],
    )(X, codebook)
    return (idx, dist)


# speedup vs baseline: 1.1402x; 1.1402x over previous
import functools

import jax
import jax.numpy as jnp
from jax.experimental import pallas as pl

_BLOCK = 2048


def _nearest_body(x_ref, cneg2_ref, c2_ref, idx_ref, dist_ref):
    x = x_ref[...]
    cneg2 = cneg2_ref[...]
    c2 = c2_ref[...]
    dots = jax.lax.dot_general(
        x, cneg2, (((1,), (1,)), ((), ())),
        preferred_element_type=jnp.float32)
    dp = dots + c2
    x2 = jnp.sum(x * x, axis=1)
    m = jnp.min(dp, axis=1)
    idx_ref[...] = jnp.argmin(dp, axis=1).astype(jnp.int32)
    dist_ref[...] = jnp.sqrt(jnp.maximum(x2 + m, 0.0))


@functools.partial(jax.jit, static_argnames=())
def kernel(X, codebook):
    n, d = X.shape
    k = codebook.shape[0]
    grid = n // _BLOCK
    cneg2 = -2.0 * codebook
    c2 = jnp.sum(codebook * codebook, axis=1)[None, :]
    idx, dist = pl.pallas_call(
        _nearest_body,
        grid=(grid,),
        in_specs=[
            pl.BlockSpec((_BLOCK, d), lambda i: (i, 0)),
            pl.BlockSpec((k, d), lambda i: (0, 0)),
            pl.BlockSpec((1, k), lambda i: (0, 0)),
        ],
        out_specs=[
            pl.BlockSpec((_BLOCK,), lambda i: (i,)),
            pl.BlockSpec((_BLOCK,), lambda i: (i,)),
        ],
        out_shape=[
            jax.ShapeDtypeStruct((n,), jnp.int32),
            jax.ShapeDtypeStruct((n,), jnp.float32),
        ],
    )(X, cneg2, c2)
    return (idx, dist)


# speedup vs baseline: 2.5405x; 2.2282x over previous
import functools

import jax
import jax.numpy as jnp
from jax.experimental import pallas as pl

_BLOCK = 2048


def _nearest_body(x_ref, cneg2_ref, c2_ref, idx_ref, dist_ref):
    x = x_ref[...]
    cneg2 = cneg2_ref[...]
    c2 = c2_ref[...]
    dots = jax.lax.dot_general(
        cneg2, x, (((1,), (1,)), ((), ())),
        preferred_element_type=jnp.float32)
    dp = dots + c2
    m = jnp.min(dp, axis=0)
    am = jnp.argmin(dp, axis=0).astype(jnp.int32)
    xx = x * x
    ones = jnp.ones((1, x.shape[1]), jnp.float32)
    x2 = jax.lax.dot_general(
        ones, xx, (((1,), (1,)), ((), ())),
        preferred_element_type=jnp.float32)[0]
    idx_ref[...] = am
    dist_ref[...] = jnp.sqrt(jnp.maximum(x2 + m, 0.0))


@functools.partial(jax.jit, static_argnames=())
def kernel(X, codebook):
    n, d = X.shape
    k = codebook.shape[0]
    grid = n // _BLOCK
    cneg2 = -2.0 * codebook
    c2 = jnp.sum(codebook * codebook, axis=1)[:, None]
    idx, dist = pl.pallas_call(
        _nearest_body,
        grid=(grid,),
        in_specs=[
            pl.BlockSpec((_BLOCK, d), lambda i: (i, 0)),
            pl.BlockSpec((k, d), lambda i: (0, 0)),
            pl.BlockSpec((k, 1), lambda i: (0, 0)),
        ],
        out_specs=[
            pl.BlockSpec((_BLOCK,), lambda i: (i,)),
            pl.BlockSpec((_BLOCK,), lambda i: (i,)),
        ],
        out_shape=[
            jax.ShapeDtypeStruct((n,), jnp.int32),
            jax.ShapeDtypeStruct((n,), jnp.float32),
        ],
    )(X, cneg2, c2)
    return (idx, dist)


# speedup vs baseline: 2.7031x; 1.0640x over previous
import functools

import jax
import jax.numpy as jnp
from jax.experimental import pallas as pl
from jax.experimental.pallas import tpu as pltpu

_BLOCK = 4096


def _nearest_body(x_ref, cneg2_ref, c2_ref, idx_ref, dist_ref):
    x = x_ref[...]
    cneg2 = cneg2_ref[...]
    c2 = c2_ref[...]
    dots = jax.lax.dot_general(
        cneg2, x, (((1,), (1,)), ((), ())),
        preferred_element_type=jnp.float32)
    dp = dots + c2
    m = jnp.min(dp, axis=0)
    am = jnp.argmin(dp, axis=0).astype(jnp.int32)
    xx = x * x
    ones = jnp.ones((1, x.shape[1]), jnp.float32)
    x2 = jax.lax.dot_general(
        ones, xx, (((1,), (1,)), ((), ())),
        preferred_element_type=jnp.float32)[0]
    idx_ref[...] = am
    dist_ref[...] = jnp.sqrt(jnp.maximum(x2 + m, 0.0))


@functools.partial(jax.jit, static_argnames=())
def kernel(X, codebook):
    n, d = X.shape
    k = codebook.shape[0]
    grid = n // _BLOCK
    cneg2 = -2.0 * codebook
    c2 = jnp.sum(codebook * codebook, axis=1)[:, None]
    idx, dist = pl.pallas_call(
        _nearest_body,
        grid=(grid,),
        in_specs=[
            pl.BlockSpec((_BLOCK, d), lambda i: (i, 0)),
            pl.BlockSpec((k, d), lambda i: (0, 0)),
            pl.BlockSpec((k, 1), lambda i: (0, 0)),
        ],
        out_specs=[
            pl.BlockSpec((_BLOCK,), lambda i: (i,)),
            pl.BlockSpec((_BLOCK,), lambda i: (i,)),
        ],
        out_shape=[
            jax.ShapeDtypeStruct((n,), jnp.int32),
            jax.ShapeDtypeStruct((n,), jnp.float32),
        ],
        compiler_params=pltpu.CompilerParams(
            dimension_semantics=("parallel",)),
    )(X, cneg2, c2)
    return (idx, dist)
